# baseline (speedup 1.0000x reference)
import functools

import jax
import jax.numpy as jnp
from jax import lax
from jax.experimental import pallas as pl
from jax.experimental.pallas import tpu as pltpu
from jax.experimental.pallas import tpu_sc as plsc

N = 10000
E = 320000
D = 128
R = 64

NC = 2
NS = 16
NW = NC * NS
CHUNK = 72
RPT = 624
REM = N - NS * RPT

E0 = 138240
E1 = E - E0
CFG0 = (0, E0 // NW, 60, 0)
CFG1 = (E0, E1 // NW, 78, 64)

BN = 2000
BE = 2560


def _silu(x):
    return x * jax.nn.sigmoid(x)


def _dot_t(x, w):
    return jax.lax.dot_general(x, w, (((1,), (1,)), ((), ())),
                               preferred_element_type=jnp.float32)


def _pack_bf16_pair(lo_f32, hi_f32):
    hi_bits = jax.lax.bitcast_convert_type(
        hi_f32.astype(jnp.bfloat16).astype(jnp.float32), jnp.int32)
    lo_bits = jax.lax.shift_right_logical(
        jax.lax.bitcast_convert_type(
            lo_f32.astype(jnp.bfloat16).astype(jnp.float32), jnp.int32),
        jnp.int32(16))
    return hi_bits | lo_bits


def _a1_body(f_ref, wi_ref, bi_ref, wj_ref, bj_ref, xi_ref, xjd_ref):
    xa = _silu(f_ref[...])
    xi_ref[...] = _silu(_dot_t(xa, wi_ref[...]) + bi_ref[...])
    xjd_ref[...] = _silu(_dot_t(xa, wj_ref[...]) + bj_ref[...])


def _stage_a1(features, wiT, bi, wjT, bj):
    grid = (N // BN,)
    return pl.pallas_call(
        _a1_body,
        grid=grid,
        in_specs=[
            pl.BlockSpec((BN, D), lambda i: (i, 0)),
            pl.BlockSpec((D, D), lambda i: (0, 0)),
            pl.BlockSpec((1, D), lambda i: (0, 0)),
            pl.BlockSpec((D, D), lambda i: (0, 0)),
            pl.BlockSpec((1, D), lambda i: (0, 0)),
        ],
        out_specs=[
            pl.BlockSpec((BN, D), lambda i: (i, 0)),
            pl.BlockSpec((BN, D), lambda i: (i, 0)),
        ],
        out_shape=[
            jax.ShapeDtypeStruct((N, D), jnp.float32),
            jax.ShapeDtypeStruct((N, D), jnp.float32),
        ],
    )(features, wiT, bi, wjT, bj)


def _a2_body(dT_ref, wglo_ref, wghi_ref, g_ref):
    dT = dT_ref[...]
    glo = jax.lax.dot_general(dT, wglo_ref[...], (((0,), (1,)), ((), ())),
                              preferred_element_type=jnp.float32)
    ghi = jax.lax.dot_general(dT, wghi_ref[...], (((0,), (1,)), ((), ())),
                              preferred_element_type=jnp.float32)
    g_ref[...] = _pack_bf16_pair(glo, ghi)


def _stage_a2(descriptorsT, Wg, ebase, ne):
    grid = (ne // BE,)
    h = D // 2
    boff = ebase // BE
    return pl.pallas_call(
        _a2_body,
        grid=grid,
        in_specs=[
            pl.BlockSpec((R, BE), lambda i: (0, boff + i)),
            pl.BlockSpec((h, R), lambda i: (0, 0)),
            pl.BlockSpec((h, R), lambda i: (1, 0)),
        ],
        out_specs=pl.BlockSpec((BE, h), lambda i: (i, 0)),
        out_shape=jax.ShapeDtypeStruct((ne, h), jnp.int32),
    )(descriptorsT, Wg, Wg)


def _sc_edge_body(cfg, g_hbm, xjd_hbm, idxj_hbm, idxi_hbm, out_hbm,
                  g_v0, g_v1, rows_v0, rows_v1, prod,
                  idxj_v0, idxj_v1, idxi_v0, idxi_v1,
                  jt, it, acc,
                  sg0, sg1, sr0, sr1, sj0, sj1, si0, si1, ss):
    ebase, epw, nchunk, tail = cfg
    c = lax.axis_index("c")
    s = lax.axis_index("s")

    gv = (g_v0, g_v1)
    rv = (rows_v0, rows_v1)
    jv = (idxj_v0, idxj_v1)
    iv = (idxi_v0, idxi_v1)
    sg = (sg0, sg1)
    sr = (sr0, sr1)
    sj = (sj0, sj1)
    si = (si0, si1)

    def _zrow(e, carry):
        for q in range(D // 16):
            prod[e, pl.ds(q * 16, 16)] = jnp.zeros((16,), jnp.float32)
        return carry
    lax.fori_loop(0, CHUNK, _zrow, 0)
    for k in range(RPT // CHUNK):
        pltpu.sync_copy(prod, acc.at[pl.ds(s * RPT + k * CHUNK, CHUNK)])
    nfull = (RPT // CHUNK) * CHUNK
    pltpu.sync_copy(prod.at[pl.ds(0, RPT - nfull)],
                    acc.at[pl.ds(s * RPT + nfull, RPT - nfull)])

    @pl.when(s == 0)
    def _zero_rem():
        pltpu.sync_copy(prod.at[pl.ds(0, REM)], acc.at[pl.ds(NS * RPT, REM)])

    plsc.subcore_barrier()

    wbase = (c * NS + s) * epw

    def _dsg(kk):
        kkw = jnp.minimum(kk, nchunk - 1)
        return pl.ds(wbase + kkw * CHUNK, CHUNK)

    def _dsi(kk):
        kkw = jnp.minimum(kk, nchunk - 1)
        return pl.ds(ebase + wbase + kkw * CHUNK, CHUNK)

    himask = jax.lax.broadcast(jnp.int32(-65536), (16,))

    def _mul_edge(gref, rref, dst_e, e):
        for q in range(D // 32):
            ds16 = pl.ds(q * 16, 16)
            gw = gref[e, ds16]
            glo = jax.lax.bitcast_convert_type(
                jax.lax.shift_left(gw, 16), jnp.float32)
            ghi = jax.lax.bitcast_convert_type(gw & himask, jnp.float32)
            rlo = rref[e, pl.ds(q * 16, 16)]
            rhi = rref[e, pl.ds(64 + q * 16, 16)]
            prod[dst_e, pl.ds(q * 16, 16)] = glo * rlo
            prod[dst_e, pl.ds(64 + q * 16, 16)] = ghi * rhi

    def _mul_chunk(b):
        def _mul(e, cc):
            _mul_edge(gv[b], rv[b], e, e)
            return cc
        lax.fori_loop(0, CHUNK, _mul, 0)

    def _step(k, b):
        nb = 1 - b
        pltpu.make_async_copy(idxj_hbm.at[_dsi(k + 1)], jv[nb], sj[nb]).wait()
        pltpu.async_copy(g_hbm.at[_dsg(k + 1)], gv[nb], sg[nb])
        pltpu.async_copy(xjd_hbm.at[jv[nb]], rv[nb], sr[nb])
        pltpu.make_async_copy(g_hbm.at[_dsg(k)], gv[b], sg[b]).wait()
        pltpu.make_async_copy(xjd_hbm.at[jv[b]], rv[b], sr[b]).wait()
        pltpu.async_copy(idxj_hbm.at[_dsi(k + 2)], jv[b], sj[b])
        pltpu.make_async_copy(prod, acc.at[iv[nb]], ss).wait()
        _mul_chunk(b)
        pltpu.make_async_copy(idxi_hbm.at[_dsi(k)], iv[b], si[b]).wait()
        pltpu.async_copy(prod, acc.at[iv[b]], ss, add=True)
        pltpu.async_copy(idxi_hbm.at[_dsi(k + 2)], iv[b], si[b])

    pltpu.async_copy(idxj_hbm.at[_dsi(0)], jv[0], sj[0])
    pltpu.async_copy(idxi_hbm.at[_dsi(0)], iv[0], si[0])
    pltpu.sync_copy(idxi_hbm.at[_dsi(1)], iv[1])
    pltpu.async_copy(prod, acc.at[iv[1]], ss, add=True)
    pltpu.async_copy(idxi_hbm.at[_dsi(1)], iv[1], si[1])
    pltpu.async_copy(idxj_hbm.at[_dsi(1)], jv[1], sj[1])
    pltpu.make_async_copy(idxj_hbm.at[_dsi(0)], jv[0], sj[0]).wait()
    pltpu.async_copy(g_hbm.at[_dsg(0)], gv[0], sg[0])
    pltpu.async_copy(xjd_hbm.at[jv[0]], rv[0], sr[0])

    def _pair(t, carry):
        _step(2 * t, 0)
        _step(2 * t + 1, 1)
        return carry
    lax.fori_loop(0, nchunk // 2, _pair, 0)

    pltpu.make_async_copy(g_hbm.at[_dsg(0)], gv[0], sg[0]).wait()
    pltpu.make_async_copy(xjd_hbm.at[jv[0]], rv[0], sr[0]).wait()
    pltpu.make_async_copy(idxj_hbm.at[_dsi(0)], jv[1], sj[1]).wait()
    pltpu.make_async_copy(idxi_hbm.at[_dsi(0)], iv[0], si[0]).wait()
    pltpu.make_async_copy(idxi_hbm.at[_dsi(0)], iv[1], si[1]).wait()
    pltpu.make_async_copy(prod, acc.at[iv[1]], ss).wait()

    if tail:
        wtail = wbase + nchunk * CHUNK
        pltpu.sync_copy(idxj_hbm.at[pl.ds(ebase + wtail, tail)], jt)
        pltpu.sync_copy(idxi_hbm.at[pl.ds(ebase + wtail, tail)], it)
        pltpu.sync_copy(g_hbm.at[pl.ds(wtail, tail)], gv[0].at[pl.ds(0, tail)])
        pltpu.async_copy(xjd_hbm.at[jt], rv[0].at[pl.ds(0, tail)], sr0).wait()

        def _mul_tail(e, cc):
            _mul_edge(gv[0], rv[0], e, e)
            return cc
        lax.fori_loop(0, tail, _mul_tail, 0)
        pltpu.sync_copy(prod.at[pl.ds(0, tail)], acc.at[it], add=True)

    plsc.subcore_barrier()

    rbase = s * RPT
    pltpu.sync_copy(acc.at[pl.ds(rbase, RPT)],
                    out_hbm.at[c, pl.ds(rbase, RPT)])

    @pl.when(s == 0)
    def _write_rem():
        pltpu.sync_copy(acc.at[pl.ds(NS * RPT, REM)],
                        out_hbm.at[c, pl.ds(NS * RPT, REM)])


def _stage_b(g, xjd, idx_j, idx_i, cfg):
    tail_buf = max(cfg[3], 8)
    mesh = plsc.VectorSubcoreMesh(core_axis_name="c", subcore_axis_name="s",
                                  num_cores=NC, num_subcores=NS)
    fn = pl.kernel(
        functools.partial(_sc_edge_body, cfg),
        out_type=jax.ShapeDtypeStruct((NC, N, D), jnp.float32),
        mesh=mesh,
        scratch_types=[
            pltpu.VMEM((CHUNK, D // 2), jnp.int32),
            pltpu.VMEM((CHUNK, D // 2), jnp.int32),
            pltpu.VMEM((CHUNK, D), jnp.float32),
            pltpu.VMEM((CHUNK, D), jnp.float32),
            pltpu.VMEM((CHUNK, D), jnp.float32),
            pltpu.VMEM((CHUNK,), jnp.int32),
            pltpu.VMEM((CHUNK,), jnp.int32),
            pltpu.VMEM((CHUNK,), jnp.int32),
            pltpu.VMEM((CHUNK,), jnp.int32),
            pltpu.VMEM((tail_buf,), jnp.int32),
            pltpu.VMEM((tail_buf,), jnp.int32),
            pltpu.VMEM_SHARED((N, D), jnp.float32),
            pltpu.SemaphoreType.DMA,
            pltpu.SemaphoreType.DMA,
            pltpu.SemaphoreType.DMA,
            pltpu.SemaphoreType.DMA,
            pltpu.SemaphoreType.DMA,
            pltpu.SemaphoreType.DMA,
            pltpu.SemaphoreType.DMA,
            pltpu.SemaphoreType.DMA,
            pltpu.SemaphoreType.DMA,
        ],
    )
    return fn(g, xjd, idx_j, idx_i)


def _c_body(xi_ref, p_ref, p2_ref, f_ref,
            riW1_ref, rib1_ref, riW2_ref, rib2_ref,
            wd_ref, bd_ref, u_ref,
            raW1_ref, rab1_ref, raW2_ref, rab2_ref,
            out_ref):
    m = xi_ref[...] + (p_ref[0] + p_ref[1]) + (p2_ref[0] + p2_ref[1])
    for i in range(riW1_ref.shape[0]):
        y = _silu(m)
        t = _silu(_dot_t(y, riW1_ref[i]) + rib1_ref[i])
        m = m + _dot_t(t, riW2_ref[i]) + rib2_ref[i]
    m = _silu(m)
    x = u_ref[...] * f_ref[...] + _dot_t(m, wd_ref[...]) + bd_ref[...]
    for i in range(raW1_ref.shape[0]):
        y = _silu(x)
        t = _silu(_dot_t(y, raW1_ref[i]) + rab1_ref[i])
        x = x + _dot_t(t, raW2_ref[i]) + rab2_ref[i]
    out_ref[...] = x


def _stage_c(xi, p, p2, features, riW1, rib1, riW2, rib2, wd, bd, u,
             raW1, rab1, raW2, rab2):
    grid = (N // BN,)
    nri = riW1.shape[0]
    nra = raW1.shape[0]
    return pl.pallas_call(
        _c_body,
        grid=grid,
        in_specs=[
            pl.BlockSpec((BN, D), lambda i: (i, 0)),
            pl.BlockSpec((NC, BN, D), lambda i: (0, i, 0)),
            pl.BlockSpec((NC, BN, D), lambda i: (0, i, 0)),
            pl.BlockSpec((BN, D), lambda i: (i, 0)),
            pl.BlockSpec((nri, D, D), lambda i: (0, 0, 0)),
            pl.BlockSpec((nri, 1, D), lambda i: (0, 0, 0)),
            pl.BlockSpec((nri, D, D), lambda i: (0, 0, 0)),
            pl.BlockSpec((nri, 1, D), lambda i: (0, 0, 0)),
            pl.BlockSpec((D, D), lambda i: (0, 0)),
            pl.BlockSpec((1, D), lambda i: (0, 0)),
            pl.BlockSpec((1, D), lambda i: (0, 0)),
            pl.BlockSpec((nra, D, D), lambda i: (0, 0, 0)),
            pl.BlockSpec((nra, 1, D), lambda i: (0, 0, 0)),
            pl.BlockSpec((nra, D, D), lambda i: (0, 0, 0)),
            pl.BlockSpec((nra, 1, D), lambda i: (0, 0, 0)),
        ],
        out_specs=pl.BlockSpec((BN, D), lambda i: (i, 0)),
        out_shape=jax.ShapeDtypeStruct((N, D), jnp.float32),
    )(xi, p, p2, features, riW1, rib1, riW2, rib2, wd, bd, u,
      raW1, rab1, raW2, rab2)


def kernel(features, descriptors, idx_i, idx_j, Wg, Wi, bi, Wj, bj,
           ri_W1, ri_b1, ri_W2, ri_b2, Wd, bd, u, ra_W1, ra_b1, ra_W2, ra_b2):
    bi2 = bi.reshape(1, D)
    bd2 = bd.reshape(1, D)
    u2 = u.reshape(1, D)
    rib1 = ri_b1.reshape(-1, 1, D)
    rib2 = ri_b2.reshape(-1, 1, D)
    rab1 = ra_b1.reshape(-1, 1, D)
    rab2 = ra_b2.reshape(-1, 1, D)

    xi, xjd = _stage_a1(features, Wi, bi2, Wj, bj.reshape(1, D))
    dT = descriptors.T
    g0 = _stage_a2(dT, Wg, 0, E0)
    p0 = _stage_b(g0, xjd, idx_j, idx_i, CFG0)
    g1 = _stage_a2(dT, Wg, E0, E1)
    p1 = _stage_b(g1, xjd, idx_j, idx_i, CFG1)
    return _stage_c(xi, p0, p1, features, ri_W1, rib1, ri_W2, rib2, Wd, bd2,
                    u2, ra_W1, rab1, ra_W2, rab2)


# speedup vs baseline: 1.0939x; 1.0939x over previous
import functools

import jax
import jax.numpy as jnp
from jax import lax
from jax.experimental import pallas as pl
from jax.experimental.pallas import tpu as pltpu
from jax.experimental.pallas import tpu_sc as plsc

N = 10000
E = 320000
D = 128
R = 64

NC = 2
NS = 16
NW = NC * NS
CHUNK = 72
RPT = 624
REM = N - NS * RPT

CFG_FULL = (0, E // NW, 138, 16)

BN = 2000
BE = 12800


def _silu(x):
    return x * jax.nn.sigmoid(x)


def _dot_t(x, w):
    return jax.lax.dot_general(x, w, (((1,), (1,)), ((), ())),
                               preferred_element_type=jnp.float32)


def _pack_bf16_pair(lo_f32, hi_f32):
    hi_bits = jax.lax.bitcast_convert_type(
        hi_f32.astype(jnp.bfloat16).astype(jnp.float32), jnp.int32)
    lo_bits = jax.lax.shift_right_logical(
        jax.lax.bitcast_convert_type(
            lo_f32.astype(jnp.bfloat16).astype(jnp.float32), jnp.int32),
        jnp.int32(16))
    return hi_bits | lo_bits


def _a1_body(f_ref, wi_ref, bi_ref, wj_ref, bj_ref, xi_ref, xjd_ref):
    xa = _silu(f_ref[...])
    xi_ref[...] = _silu(_dot_t(xa, wi_ref[...]) + bi_ref[...])
    xjd_ref[...] = _silu(_dot_t(xa, wj_ref[...]) + bj_ref[...])


def _stage_a1(features, wiT, bi, wjT, bj):
    grid = (N // BN,)
    return pl.pallas_call(
        _a1_body,
        grid=grid,
        in_specs=[
            pl.BlockSpec((BN, D), lambda i: (i, 0)),
            pl.BlockSpec((D, D), lambda i: (0, 0)),
            pl.BlockSpec((1, D), lambda i: (0, 0)),
            pl.BlockSpec((D, D), lambda i: (0, 0)),
            pl.BlockSpec((1, D), lambda i: (0, 0)),
        ],
        out_specs=[
            pl.BlockSpec((BN, D), lambda i: (i, 0)),
            pl.BlockSpec((BN, D), lambda i: (i, 0)),
        ],
        out_shape=[
            jax.ShapeDtypeStruct((N, D), jnp.float32),
            jax.ShapeDtypeStruct((N, D), jnp.float32),
        ],
    )(features, wiT, bi, wjT, bj)


def _a2_body(dT_ref, wglo_ref, wghi_ref, g_ref):
    dT = dT_ref[...]
    glo = jax.lax.dot_general(dT, wglo_ref[...], (((0,), (1,)), ((), ())),
                              preferred_element_type=jnp.float32)
    ghi = jax.lax.dot_general(dT, wghi_ref[...], (((0,), (1,)), ((), ())),
                              preferred_element_type=jnp.float32)
    g_ref[...] = _pack_bf16_pair(glo, ghi)


def _stage_a2(descriptorsT, Wg, ebase, ne):
    grid = (ne // BE,)
    h = D // 2
    boff = ebase // BE
    return pl.pallas_call(
        _a2_body,
        grid=grid,
        in_specs=[
            pl.BlockSpec((R, BE), lambda i: (0, boff + i)),
            pl.BlockSpec((h, R), lambda i: (0, 0)),
            pl.BlockSpec((h, R), lambda i: (1, 0)),
        ],
        out_specs=pl.BlockSpec((BE, h), lambda i: (i, 0)),
        out_shape=jax.ShapeDtypeStruct((ne, h), jnp.int32),
    )(descriptorsT, Wg, Wg)


def _sc_edge_body(cfg, g_hbm, xjd_hbm, idxj_hbm, idxi_hbm, out_hbm,
                  g_v0, g_v1, rows_v0, rows_v1, prod,
                  idxj_v0, idxj_v1, idxi_v0, idxi_v1,
                  jt, it, acc,
                  sg0, sg1, sr0, sr1, sj0, sj1, si0, si1, ss):
    ebase, epw, nchunk, tail = cfg
    c = lax.axis_index("c")
    s = lax.axis_index("s")

    gv = (g_v0, g_v1)
    rv = (rows_v0, rows_v1)
    jv = (idxj_v0, idxj_v1)
    iv = (idxi_v0, idxi_v1)
    sg = (sg0, sg1)
    sr = (sr0, sr1)
    sj = (sj0, sj1)
    si = (si0, si1)

    def _zrow(e, carry):
        for q in range(D // 16):
            prod[e, pl.ds(q * 16, 16)] = jnp.zeros((16,), jnp.float32)
        return carry
    lax.fori_loop(0, CHUNK, _zrow, 0)
    for k in range(RPT // CHUNK):
        pltpu.sync_copy(prod, acc.at[pl.ds(s * RPT + k * CHUNK, CHUNK)])
    nfull = (RPT // CHUNK) * CHUNK
    pltpu.sync_copy(prod.at[pl.ds(0, RPT - nfull)],
                    acc.at[pl.ds(s * RPT + nfull, RPT - nfull)])

    @pl.when(s == 0)
    def _zero_rem():
        pltpu.sync_copy(prod.at[pl.ds(0, REM)], acc.at[pl.ds(NS * RPT, REM)])

    plsc.subcore_barrier()

    wbase = (c * NS + s) * epw

    def _dsg(kk):
        kkw = jnp.minimum(kk, nchunk - 1)
        return pl.ds(wbase + kkw * CHUNK, CHUNK)

    def _dsi(kk):
        kkw = jnp.minimum(kk, nchunk - 1)
        return pl.ds(ebase + wbase + kkw * CHUNK, CHUNK)

    himask = jax.lax.broadcast(jnp.int32(-65536), (16,))

    def _mul_edge(gref, rref, dst_e, e):
        for q in range(D // 32):
            ds16 = pl.ds(q * 16, 16)
            gw = gref[e, ds16]
            glo = jax.lax.bitcast_convert_type(
                jax.lax.shift_left(gw, 16), jnp.float32)
            ghi = jax.lax.bitcast_convert_type(gw & himask, jnp.float32)
            rlo = rref[e, pl.ds(q * 16, 16)]
            rhi = rref[e, pl.ds(64 + q * 16, 16)]
            prod[dst_e, pl.ds(q * 16, 16)] = glo * rlo
            prod[dst_e, pl.ds(64 + q * 16, 16)] = ghi * rhi

    def _mul_chunk(b):
        def _mul(e, cc):
            _mul_edge(gv[b], rv[b], e, e)
            return cc
        lax.fori_loop(0, CHUNK, _mul, 0)

    def _step(k, b):
        nb = 1 - b
        pltpu.make_async_copy(idxj_hbm.at[_dsi(k + 1)], jv[nb], sj[nb]).wait()
        pltpu.async_copy(g_hbm.at[_dsg(k + 1)], gv[nb], sg[nb])
        pltpu.async_copy(xjd_hbm.at[jv[nb]], rv[nb], sr[nb])
        pltpu.make_async_copy(g_hbm.at[_dsg(k)], gv[b], sg[b]).wait()
        pltpu.make_async_copy(xjd_hbm.at[jv[b]], rv[b], sr[b]).wait()
        pltpu.async_copy(idxj_hbm.at[_dsi(k + 2)], jv[b], sj[b])
        pltpu.make_async_copy(prod, acc.at[iv[nb]], ss).wait()
        _mul_chunk(b)
        pltpu.make_async_copy(idxi_hbm.at[_dsi(k)], iv[b], si[b]).wait()
        pltpu.async_copy(prod, acc.at[iv[b]], ss, add=True)
        pltpu.async_copy(idxi_hbm.at[_dsi(k + 2)], iv[b], si[b])

    pltpu.async_copy(idxj_hbm.at[_dsi(0)], jv[0], sj[0])
    pltpu.async_copy(idxi_hbm.at[_dsi(0)], iv[0], si[0])
    pltpu.sync_copy(idxi_hbm.at[_dsi(1)], iv[1])
    pltpu.async_copy(prod, acc.at[iv[1]], ss, add=True)
    pltpu.async_copy(idxi_hbm.at[_dsi(1)], iv[1], si[1])
    pltpu.async_copy(idxj_hbm.at[_dsi(1)], jv[1], sj[1])
    pltpu.make_async_copy(idxj_hbm.at[_dsi(0)], jv[0], sj[0]).wait()
    pltpu.async_copy(g_hbm.at[_dsg(0)], gv[0], sg[0])
    pltpu.async_copy(xjd_hbm.at[jv[0]], rv[0], sr[0])

    def _pair(t, carry):
        _step(2 * t, 0)
        _step(2 * t + 1, 1)
        return carry
    lax.fori_loop(0, nchunk // 2, _pair, 0)

    pltpu.make_async_copy(g_hbm.at[_dsg(0)], gv[0], sg[0]).wait()
    pltpu.make_async_copy(xjd_hbm.at[jv[0]], rv[0], sr[0]).wait()
    pltpu.make_async_copy(idxj_hbm.at[_dsi(0)], jv[1], sj[1]).wait()
    pltpu.make_async_copy(idxi_hbm.at[_dsi(0)], iv[0], si[0]).wait()
    pltpu.make_async_copy(idxi_hbm.at[_dsi(0)], iv[1], si[1]).wait()
    pltpu.make_async_copy(prod, acc.at[iv[1]], ss).wait()

    if tail:
        wtail = wbase + nchunk * CHUNK
        pltpu.sync_copy(idxj_hbm.at[pl.ds(ebase + wtail, tail)], jt)
        pltpu.sync_copy(idxi_hbm.at[pl.ds(ebase + wtail, tail)], it)
        pltpu.sync_copy(g_hbm.at[pl.ds(wtail, tail)], gv[0].at[pl.ds(0, tail)])
        pltpu.async_copy(xjd_hbm.at[jt], rv[0].at[pl.ds(0, tail)], sr0).wait()

        def _mul_tail(e, cc):
            _mul_edge(gv[0], rv[0], e, e)
            return cc
        lax.fori_loop(0, tail, _mul_tail, 0)
        pltpu.sync_copy(prod.at[pl.ds(0, tail)], acc.at[it], add=True)

    plsc.subcore_barrier()

    rbase = s * RPT
    pltpu.sync_copy(acc.at[pl.ds(rbase, RPT)],
                    out_hbm.at[c, pl.ds(rbase, RPT)])

    @pl.when(s == 0)
    def _write_rem():
        pltpu.sync_copy(acc.at[pl.ds(NS * RPT, REM)],
                        out_hbm.at[c, pl.ds(NS * RPT, REM)])


def _stage_b(g, xjd, idx_j, idx_i, cfg):
    tail_buf = max(cfg[3], 8)
    mesh = plsc.VectorSubcoreMesh(core_axis_name="c", subcore_axis_name="s",
                                  num_cores=NC, num_subcores=NS)
    fn = pl.kernel(
        functools.partial(_sc_edge_body, cfg),
        out_type=jax.ShapeDtypeStruct((NC, N, D), jnp.float32),
        mesh=mesh,
        scratch_types=[
            pltpu.VMEM((CHUNK, D // 2), jnp.int32),
            pltpu.VMEM((CHUNK, D // 2), jnp.int32),
            pltpu.VMEM((CHUNK, D), jnp.float32),
            pltpu.VMEM((CHUNK, D), jnp.float32),
            pltpu.VMEM((CHUNK, D), jnp.float32),
            pltpu.VMEM((CHUNK,), jnp.int32),
            pltpu.VMEM((CHUNK,), jnp.int32),
            pltpu.VMEM((CHUNK,), jnp.int32),
            pltpu.VMEM((CHUNK,), jnp.int32),
            pltpu.VMEM((tail_buf,), jnp.int32),
            pltpu.VMEM((tail_buf,), jnp.int32),
            pltpu.VMEM_SHARED((N, D), jnp.float32),
            pltpu.SemaphoreType.DMA,
            pltpu.SemaphoreType.DMA,
            pltpu.SemaphoreType.DMA,
            pltpu.SemaphoreType.DMA,
            pltpu.SemaphoreType.DMA,
            pltpu.SemaphoreType.DMA,
            pltpu.SemaphoreType.DMA,
            pltpu.SemaphoreType.DMA,
            pltpu.SemaphoreType.DMA,
        ],
    )
    return fn(g, xjd, idx_j, idx_i)


def _c_body(xi_ref, p_ref, f_ref,
            riW1_ref, rib1_ref, riW2_ref, rib2_ref,
            wd_ref, bd_ref, u_ref,
            raW1_ref, rab1_ref, raW2_ref, rab2_ref,
            out_ref):
    m = xi_ref[...] + (p_ref[0] + p_ref[1])
    for i in range(riW1_ref.shape[0]):
        y = _silu(m)
        t = _silu(_dot_t(y, riW1_ref[i]) + rib1_ref[i])
        m = m + _dot_t(t, riW2_ref[i]) + rib2_ref[i]
    m = _silu(m)
    x = u_ref[...] * f_ref[...] + _dot_t(m, wd_ref[...]) + bd_ref[...]
    for i in range(raW1_ref.shape[0]):
        y = _silu(x)
        t = _silu(_dot_t(y, raW1_ref[i]) + rab1_ref[i])
        x = x + _dot_t(t, raW2_ref[i]) + rab2_ref[i]
    out_ref[...] = x


def _stage_c(xi, p, features, riW1, rib1, riW2, rib2, wd, bd, u,
             raW1, rab1, raW2, rab2):
    grid = (N // BN,)
    nri = riW1.shape[0]
    nra = raW1.shape[0]
    return pl.pallas_call(
        _c_body,
        grid=grid,
        in_specs=[
            pl.BlockSpec((BN, D), lambda i: (i, 0)),
            pl.BlockSpec((NC, BN, D), lambda i: (0, i, 0)),
            pl.BlockSpec((BN, D), lambda i: (i, 0)),
            pl.BlockSpec((nri, D, D), lambda i: (0, 0, 0)),
            pl.BlockSpec((nri, 1, D), lambda i: (0, 0, 0)),
            pl.BlockSpec((nri, D, D), lambda i: (0, 0, 0)),
            pl.BlockSpec((nri, 1, D), lambda i: (0, 0, 0)),
            pl.BlockSpec((D, D), lambda i: (0, 0)),
            pl.BlockSpec((1, D), lambda i: (0, 0)),
            pl.BlockSpec((1, D), lambda i: (0, 0)),
            pl.BlockSpec((nra, D, D), lambda i: (0, 0, 0)),
            pl.BlockSpec((nra, 1, D), lambda i: (0, 0, 0)),
            pl.BlockSpec((nra, D, D), lambda i: (0, 0, 0)),
            pl.BlockSpec((nra, 1, D), lambda i: (0, 0, 0)),
        ],
        out_specs=pl.BlockSpec((BN, D), lambda i: (i, 0)),
        out_shape=jax.ShapeDtypeStruct((N, D), jnp.float32),
    )(xi, p, features, riW1, rib1, riW2, rib2, wd, bd, u,
      raW1, rab1, raW2, rab2)


def kernel(features, descriptors, idx_i, idx_j, Wg, Wi, bi, Wj, bj,
           ri_W1, ri_b1, ri_W2, ri_b2, Wd, bd, u, ra_W1, ra_b1, ra_W2, ra_b2):
    bi2 = bi.reshape(1, D)
    bd2 = bd.reshape(1, D)
    u2 = u.reshape(1, D)
    rib1 = ri_b1.reshape(-1, 1, D)
    rib2 = ri_b2.reshape(-1, 1, D)
    rab1 = ra_b1.reshape(-1, 1, D)
    rab2 = ra_b2.reshape(-1, 1, D)

    xi, xjd = _stage_a1(features, Wi, bi2, Wj, bj.reshape(1, D))
    g = _stage_a2(descriptors.T, Wg, 0, E)
    p = _stage_b(g, xjd, idx_j, idx_i, CFG_FULL)
    return _stage_c(xi, p, features, ri_W1, rib1, ri_W2, rib2, Wd, bd2,
                    u2, ra_W1, rab1, ra_W2, rab2)
